# baseline (speedup 1.0000x reference)
import functools

import jax
import jax.numpy as jnp
from jax import lax
from jax.experimental import pallas as pl
from jax.experimental.pallas import tpu as pltpu
from jax.experimental.pallas import tpu_sc as plsc

NC = 2
NS = 16
NW = NC * NS
K = 128
RS = 4


def _sc_kernel(x_hbm, rec_hbm, g_hbm, z_hbm, z1_hbm,
               part_hbm, gpart_hbm,
               rec_v, gf_v, rows_v, acc, gacc,
               sem_t, sem_g, sem_s, sem_q,
               *, ch, n_pad, d):
    cid = lax.axis_index("c")
    sid = lax.axis_index("s")
    wid = sid * NC + cid
    rpt = n_pad // NS

    pltpu.sync_copy(z_hbm, acc.at[pl.ds(sid * rpt, rpt)])
    pltpu.sync_copy(z1_hbm, gacc.at[pl.ds(sid * rpt, rpt)])
    plsc.subcore_barrier()

    def stage_start(c):
        pltpu.async_copy(rec_hbm.at[wid, c], rec_v.at[c % RS], sem_t)
        pltpu.async_copy(g_hbm.at[wid, c], gf_v.at[c % RS], sem_t)

    def stage_wait(c):
        pltpu.make_async_copy(rec_hbm.at[wid, c], rec_v.at[c % RS],
                              sem_t).wait()
        pltpu.make_async_copy(g_hbm.at[wid, c], gf_v.at[c % RS],
                              sem_t).wait()

    def gather_start(c):
        pltpu.async_copy(x_hbm.at[rec_v.at[c % RS, 0]], rows_v.at[c % 2],
                         sem_g)

    def gather_wait(c):
        pltpu.make_async_copy(x_hbm.at[rec_v.at[c % RS, 0]],
                              rows_v.at[c % 2], sem_g).wait()

    def scatter_start(c):
        pltpu.async_copy(gf_v.at[c % RS], gacc.at[rec_v.at[c % RS, 1]],
                         sem_q, add=True)
        pltpu.async_copy(rows_v.at[c % 2], acc.at[rec_v.at[c % RS, 1]],
                         sem_s, add=True)

    def scatter_wait(c):
        pltpu.make_async_copy(gf_v.at[c % RS], gacc.at[rec_v.at[c % RS, 1]],
                              sem_q).wait()
        pltpu.make_async_copy(rows_v.at[c % 2], acc.at[rec_v.at[c % RS, 1]],
                              sem_s).wait()

    stage_start(0)
    @pl.when(ch > 1)
    def _():
        stage_start(1)
    stage_wait(0)
    gather_start(0)

    def chunk_body(c, _):
        b = c % 2

        @pl.when(c + 2 < ch)
        def _():
            stage_start(c + 2)

        @pl.when(c + 1 < ch)
        def _():
            @pl.when(c >= 1)
            def _():
                scatter_wait(c - 1)
            stage_wait(c + 1)
            gather_start(c + 1)

        gather_wait(c)

        def scale_body(jj, _):
            gv = gf_v[c % RS, pl.ds(jj * 16, 16)]
            for i in range(16):
                gb = jnp.full((16,), gv[i], jnp.float32)
                r = jj * 16 + i
                for j in range(d // 16):
                    sl = pl.ds(j * 16, 16)
                    rows_v[b, r, sl] = rows_v[b, r, sl] * gb
            return 0
        lax.fori_loop(0, K // 16, scale_body, 0)

        scatter_start(c)
        return 0

    lax.fori_loop(0, ch, chunk_body, 0)
    scatter_wait(ch - 1)
    @pl.when(ch >= 2)
    def _():
        scatter_wait(ch - 2)
    plsc.subcore_barrier()

    sl = pl.ds(sid * rpt, rpt)
    pltpu.sync_copy(acc.at[sl], part_hbm.at[cid, sl])
    pltpu.sync_copy(gacc.at[sl],
                    gpart_hbm.at[pl.ds(cid * n_pad + sid * rpt, rpt)])


def _combine_kernel(p_ref, gp_ref, x_ref, o_ref):
    gs = gp_ref[:, 0] + gp_ref[:, 1]
    o_ref[...] = p_ref[0] + p_ref[1] - (1.0 + gs)[:, None] * x_ref[...]


@jax.jit
def kernel(x, g, edge_index):
    n, d = x.shape
    e = g.shape[0]
    dst = edge_index[0]
    src = edge_index[1]

    ch = -(-e // (NW * K))
    e_pad = NW * K * ch
    n_pad = -(-n // (NS * K)) * (NS * K)
    rpt = n_pad // NS

    pad = e_pad - e
    src_p = jnp.concatenate([src, jnp.zeros((pad,), jnp.int32)]).reshape(NW, ch, K)
    dst_p = jnp.concatenate([dst, jnp.zeros((pad,), jnp.int32)]).reshape(NW, ch, K)
    g_p = jnp.concatenate([g, jnp.zeros((pad,), jnp.float32)]).reshape(NW, ch, K)
    rec = jnp.stack([src_p, dst_p], axis=2)
    z = jnp.zeros((rpt, d), jnp.float32)
    z1 = jnp.zeros((rpt,), jnp.float32)

    mesh = plsc.VectorSubcoreMesh(core_axis_name="c", subcore_axis_name="s",
                                  num_cores=NC, num_subcores=NS)
    part, gpart = pl.kernel(
        functools.partial(_sc_kernel, ch=ch, n_pad=n_pad, d=d),
        out_type=(jax.ShapeDtypeStruct((NC, n_pad, d), jnp.float32),
                  jax.ShapeDtypeStruct((NC * n_pad,), jnp.float32)),
        mesh=mesh,
        scratch_types=[
            pltpu.VMEM((RS, 2, K), jnp.int32),
            pltpu.VMEM((RS, K), jnp.float32),
            pltpu.VMEM((2, K, d), jnp.float32),
            pltpu.VMEM_SHARED((n_pad, d), jnp.float32),
            pltpu.VMEM_SHARED((n_pad,), jnp.float32),
            pltpu.SemaphoreType.DMA,
            pltpu.SemaphoreType.DMA,
            pltpu.SemaphoreType.DMA,
            pltpu.SemaphoreType.DMA,
        ],
    )(x, rec, g_p, z, z1)

    rb = 80
    out = pl.pallas_call(
        _combine_kernel,
        grid=(n // rb,),
        in_specs=[
            pl.BlockSpec((NC, rb, d), lambda i: (0, i, 0)),
            pl.BlockSpec((rb, NC), lambda i: (i, 0)),
            pl.BlockSpec((rb, d), lambda i: (i, 0)),
        ],
        out_specs=pl.BlockSpec((rb, d), lambda i: (i, 0)),
        out_shape=jax.ShapeDtypeStruct((n, d), jnp.float32),
    )(part, gpart.reshape(NC, n_pad).T, x)

    return out


# speedup vs baseline: 1.2832x; 1.2832x over previous
import functools

import jax
import jax.numpy as jnp
from jax import lax
from jax.experimental import pallas as pl
from jax.experimental.pallas import tpu as pltpu
from jax.experimental.pallas import tpu_sc as plsc

NC = 2
NS = 16
NW = NC * NS
K = 128
RS = 3


def _sc_kernel(x_hbm, rec_hbm, gg_hbm, z_hbm, z1_hbm,
               part_hbm, gpart_hbm,
               rec_v, gg_v, rows_v, acc, gacc,
               sem_t, sem_g, sem_s, sem_q,
               *, ch, n_pad, d):
    cid = lax.axis_index("c")
    sid = lax.axis_index("s")
    wid = sid * NC + cid
    rpt = n_pad // NS

    pltpu.sync_copy(z_hbm, acc.at[pl.ds(sid * rpt, rpt)])
    pltpu.sync_copy(z1_hbm, gacc.at[pl.ds(sid * rpt, rpt)])
    plsc.subcore_barrier()

    def stage_start(c):
        pltpu.async_copy(rec_hbm.at[wid, c], rec_v.at[c % RS], sem_t)
        pltpu.async_copy(gg_hbm.at[wid, c], gg_v.at[c % RS], sem_t)

    def stage_wait(c):
        pltpu.make_async_copy(rec_hbm.at[wid, c], rec_v.at[c % RS],
                              sem_t).wait()
        pltpu.make_async_copy(gg_hbm.at[wid, c], gg_v.at[c % RS],
                              sem_t).wait()

    def gather_start(c):
        pltpu.async_copy(x_hbm.at[rec_v.at[c % RS, 0]], rows_v.at[c % 2],
                         sem_g)

    def gather_wait(c):
        pltpu.make_async_copy(x_hbm.at[rec_v.at[c % RS, 0]],
                              rows_v.at[c % 2], sem_g).wait()

    def scatter_start(c):
        pltpu.async_copy(gg_v.at[c % RS, 0], gacc.at[rec_v.at[c % RS, 1]],
                         sem_q, add=True)
        pltpu.async_copy(rows_v.at[c % 2], acc.at[rec_v.at[c % RS, 1]],
                         sem_s, add=True)

    def scatter_wait(c):
        pltpu.make_async_copy(gg_v.at[c % RS, 0],
                              gacc.at[rec_v.at[c % RS, 1]], sem_q).wait()
        pltpu.make_async_copy(rows_v.at[c % 2], acc.at[rec_v.at[c % RS, 1]],
                              sem_s).wait()

    stage_start(0)
    @pl.when(ch > 1)
    def _():
        stage_start(1)
    stage_wait(0)
    gather_start(0)

    def chunk_body(c, _):
        b = c % 2

        @pl.when(c >= 1)
        def _():
            scatter_wait(c - 1)

        @pl.when(c + 2 < ch)
        def _():
            stage_start(c + 2)

        @pl.when(c + 1 < ch)
        def _():
            stage_wait(c + 1)
            gather_start(c + 1)

        gather_wait(c)

        def scale_body(jj, _):
            for rr in range(8):
                gb = gg_v[c % RS, 1 + jj, pl.ds(rr * 16, 16)]
                r = jj * 8 + rr
                for j in range(d // 16):
                    sl = pl.ds(j * 16, 16)
                    rows_v[b, r, sl] = rows_v[b, r, sl] * gb
            return 0
        lax.fori_loop(0, K // 8, scale_body, 0)

        scatter_start(c)
        return 0

    lax.fori_loop(0, ch, chunk_body, 0)
    scatter_wait(ch - 1)
    plsc.subcore_barrier()

    sl = pl.ds(sid * rpt, rpt)
    pltpu.sync_copy(acc.at[sl], part_hbm.at[cid, sl])
    pltpu.sync_copy(gacc.at[sl],
                    gpart_hbm.at[pl.ds(cid * n_pad + sid * rpt, rpt)])


def _combine_kernel(p_ref, gp_ref, x_ref, o_ref):
    gs = gp_ref[:, 0] + gp_ref[:, 1]
    o_ref[...] = p_ref[0] + p_ref[1] - (1.0 + gs)[:, None] * x_ref[...]


@jax.jit
def kernel(x, g, edge_index):
    n, d = x.shape
    e = g.shape[0]
    dst = edge_index[0]
    src = edge_index[1]

    ch = -(-e // (NW * K))
    e_pad = NW * K * ch
    n_pad = -(-n // (NS * K)) * (NS * K)
    rpt = n_pad // NS

    pad = e_pad - e
    src_p = jnp.concatenate([src, jnp.zeros((pad,), jnp.int32)]).reshape(NW, ch, K)
    dst_p = jnp.concatenate([dst, jnp.zeros((pad,), jnp.int32)]).reshape(NW, ch, K)
    g_p = jnp.concatenate([g, jnp.zeros((pad,), jnp.float32)]).reshape(NW, ch, K)
    rec = jnp.stack([src_p, dst_p], axis=2)
    gbc = jnp.broadcast_to(g_p[..., None], (NW, ch, K, 16))
    gg_p = jnp.concatenate([g_p[:, :, None, :],
                            gbc.reshape(NW, ch, 16, K)], axis=2)
    z = jnp.zeros((rpt, d), jnp.float32)
    z1 = jnp.zeros((rpt,), jnp.float32)

    mesh = plsc.VectorSubcoreMesh(core_axis_name="c", subcore_axis_name="s",
                                  num_cores=NC, num_subcores=NS)
    part, gpart = pl.kernel(
        functools.partial(_sc_kernel, ch=ch, n_pad=n_pad, d=d),
        out_type=(jax.ShapeDtypeStruct((NC, n_pad, d), jnp.float32),
                  jax.ShapeDtypeStruct((NC * n_pad,), jnp.float32)),
        mesh=mesh,
        scratch_types=[
            pltpu.VMEM((RS, 2, K), jnp.int32),
            pltpu.VMEM((RS, 17, K), jnp.float32),
            pltpu.VMEM((2, K, d), jnp.float32),
            pltpu.VMEM_SHARED((n_pad, d), jnp.float32),
            pltpu.VMEM_SHARED((n_pad,), jnp.float32),
            pltpu.SemaphoreType.DMA,
            pltpu.SemaphoreType.DMA,
            pltpu.SemaphoreType.DMA,
            pltpu.SemaphoreType.DMA,
        ],
    )(x, rec, gg_p, z, z1)

    rb = 80
    out = pl.pallas_call(
        _combine_kernel,
        grid=(n // rb,),
        in_specs=[
            pl.BlockSpec((NC, rb, d), lambda i: (0, i, 0)),
            pl.BlockSpec((rb, NC), lambda i: (i, 0)),
            pl.BlockSpec((rb, d), lambda i: (i, 0)),
        ],
        out_specs=pl.BlockSpec((rb, d), lambda i: (i, 0)),
        out_shape=jax.ShapeDtypeStruct((n, d), jnp.float32),
    )(part, gpart.reshape(NC, n_pad).T, x)

    return out


# speedup vs baseline: 1.2846x; 1.0011x over previous
import functools

import jax
import jax.numpy as jnp
from jax import lax
from jax.experimental import pallas as pl
from jax.experimental.pallas import tpu as pltpu
from jax.experimental.pallas import tpu_sc as plsc

NC = 2
NS = 16
NW = NC * NS
K = 128
RS = 3


def _sc_kernel(x_hbm, rec_hbm, gg_hbm, z_hbm, z1_hbm,
               part_hbm, gpart_hbm,
               rec_v, gg_v, rows_v, acc, gacc,
               sem_t, sem_g, sem_s, sem_q,
               *, ch, n_pad, d):
    cid = lax.axis_index("c")
    sid = lax.axis_index("s")
    wid = sid * NC + cid
    rpt = n_pad // NS

    pltpu.sync_copy(z_hbm, acc.at[pl.ds(sid * rpt, rpt)])
    pltpu.sync_copy(z1_hbm, gacc.at[pl.ds(sid * rpt, rpt)])
    plsc.subcore_barrier()

    def stage_start(c):
        pltpu.async_copy(rec_hbm.at[wid, c], rec_v.at[c % RS], sem_t)
        pltpu.async_copy(gg_hbm.at[wid, c], gg_v.at[c % RS], sem_t)

    def stage_wait(c):
        pltpu.make_async_copy(rec_hbm.at[wid, c], rec_v.at[c % RS],
                              sem_t).wait()
        pltpu.make_async_copy(gg_hbm.at[wid, c], gg_v.at[c % RS],
                              sem_t).wait()

    def gather_start(c):
        pltpu.async_copy(x_hbm.at[rec_v.at[c % RS, 0]], rows_v.at[c % 2],
                         sem_g)

    def gather_wait(c):
        pltpu.make_async_copy(x_hbm.at[rec_v.at[c % RS, 0]],
                              rows_v.at[c % 2], sem_g).wait()

    def scatter_start(c):
        pltpu.async_copy(gg_v.at[c % RS, 0], gacc.at[rec_v.at[c % RS, 1]],
                         sem_q, add=True)
        pltpu.async_copy(rows_v.at[c % 2], acc.at[rec_v.at[c % RS, 1]],
                         sem_s, add=True)

    def scatter_wait(c):
        pltpu.make_async_copy(gg_v.at[c % RS, 0],
                              gacc.at[rec_v.at[c % RS, 1]], sem_q).wait()
        pltpu.make_async_copy(rows_v.at[c % 2], acc.at[rec_v.at[c % RS, 1]],
                              sem_s).wait()

    stage_start(0)
    @pl.when(ch > 1)
    def _():
        stage_start(1)
    stage_wait(0)
    gather_start(0)

    def chunk_body(c, _):
        b = c % 2

        @pl.when(c >= 1)
        def _():
            scatter_wait(c - 1)

        @pl.when(c + 1 < ch)
        def _():
            stage_wait(c + 1)
            gather_start(c + 1)

        @pl.when(c + 2 < ch)
        def _():
            stage_start(c + 2)

        gather_wait(c)

        def scale_body(jj, _):
            for rr in range(8):
                gb = gg_v[c % RS, 1 + jj, pl.ds(rr * 16, 16)]
                r = jj * 8 + rr
                for j in range(d // 16):
                    sl = pl.ds(j * 16, 16)
                    rows_v[b, r, sl] = rows_v[b, r, sl] * gb
            return 0
        lax.fori_loop(0, K // 8, scale_body, 0)

        scatter_start(c)
        return 0

    lax.fori_loop(0, ch, chunk_body, 0)
    scatter_wait(ch - 1)
    plsc.subcore_barrier()

    sl = pl.ds(sid * rpt, rpt)
    pltpu.sync_copy(acc.at[sl], part_hbm.at[cid, sl])
    pltpu.sync_copy(gacc.at[sl],
                    gpart_hbm.at[pl.ds(cid * n_pad + sid * rpt, rpt)])


def _combine_kernel(p_ref, gp_ref, x_ref, o_ref):
    gs = gp_ref[:, 0] + gp_ref[:, 1]
    o_ref[...] = p_ref[0] + p_ref[1] - (1.0 + gs)[:, None] * x_ref[...]


@jax.jit
def kernel(x, g, edge_index):
    n, d = x.shape
    e = g.shape[0]
    dst = edge_index[0]
    src = edge_index[1]

    ch = -(-e // (NW * K))
    e_pad = NW * K * ch
    n_pad = -(-n // (NS * K)) * (NS * K)
    rpt = n_pad // NS

    pad = e_pad - e
    src_p = jnp.concatenate([src, jnp.zeros((pad,), jnp.int32)]).reshape(NW, ch, K)
    dst_p = jnp.concatenate([dst, jnp.zeros((pad,), jnp.int32)]).reshape(NW, ch, K)
    g_p = jnp.concatenate([g, jnp.zeros((pad,), jnp.float32)]).reshape(NW, ch, K)
    rec = jnp.stack([src_p, dst_p], axis=2)
    gbc = jnp.broadcast_to(g_p[..., None], (NW, ch, K, 16))
    gg_p = jnp.concatenate([g_p[:, :, None, :],
                            gbc.reshape(NW, ch, 16, K)], axis=2)
    z = jnp.zeros((rpt, d), jnp.float32)
    z1 = jnp.zeros((rpt,), jnp.float32)

    mesh = plsc.VectorSubcoreMesh(core_axis_name="c", subcore_axis_name="s",
                                  num_cores=NC, num_subcores=NS)
    part, gpart = pl.kernel(
        functools.partial(_sc_kernel, ch=ch, n_pad=n_pad, d=d),
        out_type=(jax.ShapeDtypeStruct((NC, n_pad, d), jnp.float32),
                  jax.ShapeDtypeStruct((NC * n_pad,), jnp.float32)),
        mesh=mesh,
        scratch_types=[
            pltpu.VMEM((RS, 2, K), jnp.int32),
            pltpu.VMEM((RS, 17, K), jnp.float32),
            pltpu.VMEM((2, K, d), jnp.float32),
            pltpu.VMEM_SHARED((n_pad, d), jnp.float32),
            pltpu.VMEM_SHARED((n_pad,), jnp.float32),
            pltpu.SemaphoreType.DMA,
            pltpu.SemaphoreType.DMA,
            pltpu.SemaphoreType.DMA,
            pltpu.SemaphoreType.DMA,
        ],
    )(x, rec, gg_p, z, z1)

    rb = 80
    out = pl.pallas_call(
        _combine_kernel,
        grid=(n // rb,),
        in_specs=[
            pl.BlockSpec((NC, rb, d), lambda i: (0, i, 0)),
            pl.BlockSpec((rb, NC), lambda i: (i, 0)),
            pl.BlockSpec((rb, d), lambda i: (i, 0)),
        ],
        out_specs=pl.BlockSpec((rb, d), lambda i: (i, 0)),
        out_shape=jax.ShapeDtypeStruct((n, d), jnp.float32),
    )(part, gpart.reshape(NC, n_pad).T, x)

    return out
